# Initial kernel scaffold; baseline (speedup 1.0000x reference)
import jax
import jax.numpy as jnp
from jax.experimental import pallas as pl


def kernel(gt_image, pre_image):
    raise NotImplementedError("write your pallas kernel here")


# speedup vs baseline: 27.3817x; 27.3817x over previous
import functools

import jax
import jax.numpy as jnp
from jax import lax
from jax.experimental import pallas as pl
from jax.experimental.pallas import tpu as pltpu
from jax.experimental.pallas import tpu_sc as plsc

NUM_CLASS = 19
NBINS = NUM_CLASS * NUM_CLASS
BINS_PAD = 368
LANES = 16

N_TOTAL = 16 * 512 * 512
NC = 2
NS = 16
NW = NC * NS
N_PER_W = N_TOTAL // NW
CHUNK = 16384
N_CHUNKS = N_PER_W // CHUNK
VECS_PER_CHUNK = CHUNK // LANES


def _sc_body(gt_hbm, pre_hbm, out_hbm, gt_buf, pre_buf, hist, hist1d):
  wid = lax.axis_index("s") * NC + lax.axis_index("c")
  base = wid * N_PER_W

  lane = jnp.arange(LANES, dtype=jnp.int32)
  lane_base = lane * BINS_PAD
  ones = jnp.ones((LANES,), jnp.float32)
  zeros = jnp.zeros((LANES,), jnp.float32)

  def zero_body(k, _):
    hist[pl.ds(k * LANES, LANES)] = zeros
    return 0
  lax.fori_loop(0, (LANES * BINS_PAD) // LANES, zero_body, 0)

  def chunk_body(c, _):
    off = base + c * CHUNK
    pltpu.sync_copy(gt_hbm.at[pl.ds(off, CHUNK)], gt_buf)
    pltpu.sync_copy(pre_hbm.at[pl.ds(off, CHUNK)], pre_buf)

    def vec_body(i, _):
      g = gt_buf[pl.ds(i * LANES, LANES)]
      p = pre_buf[pl.ds(i * LANES, LANES)]
      idx = g * NUM_CLASS + p + lane_base
      plsc.addupdate_scatter(hist, [idx], ones)
      return 0
    lax.fori_loop(0, VECS_PER_CHUNK, vec_body, 0)
    return 0
  lax.fori_loop(0, N_CHUNKS, chunk_body, 0)

  def col_body(cc, _):
    def lane_red(l, acc):
      return acc + hist[pl.ds(l * BINS_PAD + cc * LANES, LANES)]
    acc = lax.fori_loop(0, LANES, lane_red, zeros)
    hist1d[pl.ds(cc * LANES, LANES)] = acc
    return 0
  lax.fori_loop(0, BINS_PAD // LANES, col_body, 0)

  pltpu.sync_copy(hist1d, out_hbm.at[wid])


@jax.jit
def _confusion(gt_flat, pre_flat):
  mesh = plsc.VectorSubcoreMesh(core_axis_name="c", subcore_axis_name="s")
  partials = pl.kernel(
      _sc_body,
      out_type=jax.ShapeDtypeStruct((NW, BINS_PAD), jnp.float32),
      mesh=mesh,
      compiler_params=pltpu.CompilerParams(needs_layout_passes=False),
      scratch_types=[
          pltpu.VMEM((CHUNK,), jnp.int32),
          pltpu.VMEM((CHUNK,), jnp.int32),
          pltpu.VMEM((LANES * BINS_PAD,), jnp.float32),
          pltpu.VMEM((BINS_PAD,), jnp.float32),
      ],
  )(gt_flat, pre_flat)
  return partials.sum(axis=0)[:NBINS].reshape(NUM_CLASS, NUM_CLASS)


def kernel(gt_image, pre_image):
  gt_flat = gt_image.reshape(-1)
  pre_flat = pre_image.reshape(-1)
  return _confusion(gt_flat, pre_flat)


# speedup vs baseline: 42.5984x; 1.5557x over previous
import functools

import jax
import jax.numpy as jnp
from jax import lax
from jax.experimental import pallas as pl
from jax.experimental.pallas import tpu as pltpu
from jax.experimental.pallas import tpu_sc as plsc

NUM_CLASS = 19
NBINS = NUM_CLASS * NUM_CLASS
BINS_PAD = 368
LANES = 16

N_TOTAL = 16 * 512 * 512
NC = 2
NS = 16
NW = NC * NS
N_PER_W = N_TOTAL // NW
CHUNK = 16384
N_CHUNKS = N_PER_W // CHUNK
VECS_PER_CHUNK = CHUNK // LANES


def _sc_body(gt_hbm, pre_hbm, out_hbm, gt_buf, pre_buf, hist, hist1d):
  wid = lax.axis_index("s") * NC + lax.axis_index("c")
  base = wid * N_PER_W

  lane = jnp.arange(LANES, dtype=jnp.int32)
  lane_base = lane * BINS_PAD
  ones = jnp.ones((LANES,), jnp.float32)
  zeros = jnp.zeros((LANES,), jnp.float32)

  def zero_body(k, _):
    hist[pl.ds(k * LANES, LANES)] = zeros
    return 0
  lax.fori_loop(0, (LANES * BINS_PAD) // LANES, zero_body, 0)

  def chunk_body(c, _):
    off = base + c * CHUNK
    pltpu.sync_copy(gt_hbm.at[pl.ds(off, CHUNK)], gt_buf)
    pltpu.sync_copy(pre_hbm.at[pl.ds(off, CHUNK)], pre_buf)

    @plsc.parallel_loop(0, VECS_PER_CHUNK, unroll=8)
    def vec_body(i):
      g = gt_buf[pl.ds(i * LANES, LANES)]
      p = pre_buf[pl.ds(i * LANES, LANES)]
      idx = g * NUM_CLASS + p + lane_base
      plsc.addupdate_scatter(hist, [idx], ones)
    return 0
  lax.fori_loop(0, N_CHUNKS, chunk_body, 0)

  def col_body(cc, _):
    def lane_red(l, acc):
      return acc + hist[pl.ds(l * BINS_PAD + cc * LANES, LANES)]
    acc = lax.fori_loop(0, LANES, lane_red, zeros)
    hist1d[pl.ds(cc * LANES, LANES)] = acc
    return 0
  lax.fori_loop(0, BINS_PAD // LANES, col_body, 0)

  pltpu.sync_copy(hist1d, out_hbm.at[wid])


@jax.jit
def _confusion(gt_flat, pre_flat):
  mesh = plsc.VectorSubcoreMesh(core_axis_name="c", subcore_axis_name="s")
  partials = pl.kernel(
      _sc_body,
      out_type=jax.ShapeDtypeStruct((NW, BINS_PAD), jnp.float32),
      mesh=mesh,
      compiler_params=pltpu.CompilerParams(needs_layout_passes=False),
      scratch_types=[
          pltpu.VMEM((CHUNK,), jnp.int32),
          pltpu.VMEM((CHUNK,), jnp.int32),
          pltpu.VMEM((LANES * BINS_PAD,), jnp.float32),
          pltpu.VMEM((BINS_PAD,), jnp.float32),
      ],
  )(gt_flat, pre_flat)
  return partials.sum(axis=0)[:NBINS].reshape(NUM_CLASS, NUM_CLASS)


def kernel(gt_image, pre_image):
  gt_flat = gt_image.reshape(-1)
  pre_flat = pre_image.reshape(-1)
  return _confusion(gt_flat, pre_flat)


# speedup vs baseline: 52.9760x; 1.2436x over previous
import functools

import jax
import jax.numpy as jnp
from jax import lax
from jax.experimental import pallas as pl
from jax.experimental.pallas import tpu as pltpu
from jax.experimental.pallas import tpu_sc as plsc

NUM_CLASS = 19
NBINS = NUM_CLASS * NUM_CLASS
BINS_PAD = 368
LANES = 16

N_TOTAL = 16 * 512 * 512
NC = 2
NS = 16
NW = NC * NS
N_PER_W = N_TOTAL // NW
CHUNK = 16384
N_CHUNKS = N_PER_W // CHUNK
VECS_PER_CHUNK = CHUNK // LANES


def _sc_body(gt_hbm, pre_hbm, out_hbm, gt_buf, pre_buf, hist, hist1d,
             sem_g0, sem_g1, sem_p0, sem_p1):
  wid = lax.axis_index("s") * NC + lax.axis_index("c")
  base = wid * N_PER_W

  lane = jnp.arange(LANES, dtype=jnp.int32)
  lane_base = lane * BINS_PAD
  ones = jnp.ones((LANES,), jnp.float32)
  zeros = jnp.zeros((LANES,), jnp.float32)

  sem_g = (sem_g0, sem_g1)
  sem_p = (sem_p0, sem_p1)

  def issue(c):
    b = c % 2
    off = base + c * CHUNK
    hg = pltpu.async_copy(gt_hbm.at[pl.ds(off, CHUNK)],
                          gt_buf.at[pl.ds(b * CHUNK, CHUNK)], sem_g[b])
    hp = pltpu.async_copy(pre_hbm.at[pl.ds(off, CHUNK)],
                          pre_buf.at[pl.ds(b * CHUNK, CHUNK)], sem_p[b])
    return hg, hp

  handles = [None, None]
  handles[0] = issue(0)

  def zero_body(k, _):
    hist[pl.ds(k * LANES, LANES)] = zeros
    return 0
  lax.fori_loop(0, (LANES * BINS_PAD) // LANES, zero_body, 0)

  for c in range(N_CHUNKS):
    b = c % 2
    if c + 1 < N_CHUNKS:
      handles[(c + 1) % 2] = issue(c + 1)
    hg, hp = handles[b]
    hg.wait()
    hp.wait()
    boff = b * CHUNK

    @plsc.parallel_loop(0, VECS_PER_CHUNK, unroll=8)
    def vec_body(i, boff=boff):
      g = gt_buf[pl.ds(boff + i * LANES, LANES)]
      p = pre_buf[pl.ds(boff + i * LANES, LANES)]
      idx = g * NUM_CLASS + p + lane_base
      plsc.addupdate_scatter(hist, [idx], ones)

  def col_body(cc, _):
    def lane_red(l, acc):
      return acc + hist[pl.ds(l * BINS_PAD + cc * LANES, LANES)]
    acc = lax.fori_loop(0, LANES, lane_red, zeros)
    hist1d[pl.ds(cc * LANES, LANES)] = acc
    return 0
  lax.fori_loop(0, BINS_PAD // LANES, col_body, 0)

  pltpu.sync_copy(hist1d, out_hbm.at[wid])


@jax.jit
def _confusion(gt_flat, pre_flat):
  mesh = plsc.VectorSubcoreMesh(core_axis_name="c", subcore_axis_name="s")
  partials = pl.kernel(
      _sc_body,
      out_type=jax.ShapeDtypeStruct((NW, BINS_PAD), jnp.float32),
      mesh=mesh,
      compiler_params=pltpu.CompilerParams(needs_layout_passes=False),
      scratch_types=[
          pltpu.VMEM((2 * CHUNK,), jnp.int32),
          pltpu.VMEM((2 * CHUNK,), jnp.int32),
          pltpu.VMEM((LANES * BINS_PAD,), jnp.float32),
          pltpu.VMEM((BINS_PAD,), jnp.float32),
          pltpu.SemaphoreType.DMA,
          pltpu.SemaphoreType.DMA,
          pltpu.SemaphoreType.DMA,
          pltpu.SemaphoreType.DMA,
      ],
  )(gt_flat, pre_flat)
  return partials.sum(axis=0)[:NBINS].reshape(NUM_CLASS, NUM_CLASS)


def kernel(gt_image, pre_image):
  gt_flat = gt_image.reshape(-1)
  pre_flat = pre_image.reshape(-1)
  return _confusion(gt_flat, pre_flat)


# speedup vs baseline: 54.1421x; 1.0220x over previous
import functools

import jax
import jax.numpy as jnp
from jax import lax
from jax.experimental import pallas as pl
from jax.experimental.pallas import tpu as pltpu
from jax.experimental.pallas import tpu_sc as plsc

NUM_CLASS = 19
NBINS = NUM_CLASS * NUM_CLASS
BINS_PAD = 368
LANES = 16

N_TOTAL = 16 * 512 * 512
NC = 2
NS = 16
NW = NC * NS
N_PER_W = N_TOTAL // NW
CHUNK = 16384
N_CHUNKS = N_PER_W // CHUNK
VECS_PER_CHUNK = CHUNK // LANES


def _sc_body(gt_hbm, pre_hbm, out_hbm, gt_buf, pre_buf, hist, hist1d,
             sem_g0, sem_g1, sem_p0, sem_p1):
  wid = lax.axis_index("s") * NC + lax.axis_index("c")
  base = wid * N_PER_W

  lane = jnp.arange(LANES, dtype=jnp.int32)
  ones = jnp.ones((LANES,), jnp.float32)
  zeros = jnp.zeros((LANES,), jnp.float32)

  sem_g = (sem_g0, sem_g1)
  sem_p = (sem_p0, sem_p1)

  def issue(c):
    b = c % 2
    off = base + c * CHUNK
    hg = pltpu.async_copy(gt_hbm.at[pl.ds(off, CHUNK)],
                          gt_buf.at[pl.ds(b * CHUNK, CHUNK)], sem_g[b])
    hp = pltpu.async_copy(pre_hbm.at[pl.ds(off, CHUNK)],
                          pre_buf.at[pl.ds(b * CHUNK, CHUNK)], sem_p[b])
    return hg, hp

  handles = [None, None]
  handles[0] = issue(0)

  def zero_body(k, _):
    hist[pl.ds(k * LANES, LANES)] = zeros
    return 0
  lax.fori_loop(0, (LANES * BINS_PAD) // LANES, zero_body, 0)

  for c in range(N_CHUNKS):
    b = c % 2
    if c + 1 < N_CHUNKS:
      handles[(c + 1) % 2] = issue(c + 1)
    hg, hp = handles[b]
    hg.wait()
    hp.wait()
    boff = b * CHUNK

    @plsc.parallel_loop(0, VECS_PER_CHUNK, unroll=8)
    def vec_body(i, boff=boff):
      g = gt_buf[pl.ds(boff + i * LANES, LANES)]
      p = pre_buf[pl.ds(boff + i * LANES, LANES)]
      idx = (g * NUM_CLASS + p) * LANES + lane
      plsc.addupdate_scatter(hist, [idx], ones)

  def scan_body(bb, _):
    v = hist[pl.ds(bb * LANES, LANES)]
    hist[pl.ds(bb * LANES, LANES)] = plsc.cumsum(v)
    return 0
  lax.fori_loop(0, BINS_PAD, scan_body, 0)

  def col_body(cc, _):
    idx = (cc * LANES + lane) * LANES + (LANES - 1)
    hist1d[pl.ds(cc * LANES, LANES)] = plsc.load_gather(hist, [idx])
    return 0
  lax.fori_loop(0, BINS_PAD // LANES, col_body, 0)

  pltpu.sync_copy(hist1d, out_hbm.at[wid])


@jax.jit
def _confusion(gt_flat, pre_flat):
  mesh = plsc.VectorSubcoreMesh(core_axis_name="c", subcore_axis_name="s")
  partials = pl.kernel(
      _sc_body,
      out_type=jax.ShapeDtypeStruct((NW, BINS_PAD), jnp.float32),
      mesh=mesh,
      compiler_params=pltpu.CompilerParams(needs_layout_passes=False),
      scratch_types=[
          pltpu.VMEM((2 * CHUNK,), jnp.int32),
          pltpu.VMEM((2 * CHUNK,), jnp.int32),
          pltpu.VMEM((LANES * BINS_PAD,), jnp.float32),
          pltpu.VMEM((BINS_PAD,), jnp.float32),
          pltpu.SemaphoreType.DMA,
          pltpu.SemaphoreType.DMA,
          pltpu.SemaphoreType.DMA,
          pltpu.SemaphoreType.DMA,
      ],
  )(gt_flat, pre_flat)
  return partials.sum(axis=0)[:NBINS].reshape(NUM_CLASS, NUM_CLASS)


def kernel(gt_image, pre_image):
  gt_flat = gt_image.reshape(-1)
  pre_flat = pre_image.reshape(-1)
  return _confusion(gt_flat, pre_flat)


# speedup vs baseline: 95.7765x; 1.7690x over previous
import functools

import jax
import jax.numpy as jnp
from jax import lax
from jax.experimental import pallas as pl
from jax.experimental.pallas import tpu as pltpu
from jax.experimental.pallas import tpu_sc as plsc

NUM_CLASS = 19
NBINS = NUM_CLASS * NUM_CLASS
BINS_PAD = 368
LANES = 16

N_TOTAL = 16 * 512 * 512
NC = 2
NS = 16
NW = NC * NS
IMG_H = 512
IMG_W = 512
ROWS_PER_W = 256
ROWS_PER_CHUNK = 32
CHUNK = ROWS_PER_CHUNK * IMG_W
N_CHUNKS = ROWS_PER_W // ROWS_PER_CHUNK
VECS_PER_CHUNK = CHUNK // LANES
VECS_PER_ROW = IMG_W // LANES


def _sc_body(gt_hbm, pre_hbm, out_hbm, gt_buf, pre_buf, hist, hist1d,
             sem_g0, sem_g1, sem_p0, sem_p1):
  wid = lax.axis_index("s") * NC + lax.axis_index("c")
  img = wid // 2
  row_base = (wid % 2) * ROWS_PER_W

  lane = jnp.arange(LANES, dtype=jnp.int32)
  ones = jnp.ones((LANES,), jnp.float32)
  zeros = jnp.zeros((LANES,), jnp.float32)

  sem_g = (sem_g0, sem_g1)
  sem_p = (sem_p0, sem_p1)

  def issue(c):
    b = c % 2
    r0 = row_base + c * ROWS_PER_CHUNK
    dst = pl.ds(b * ROWS_PER_CHUNK, ROWS_PER_CHUNK)
    hg = pltpu.async_copy(gt_hbm.at[img, pl.ds(r0, ROWS_PER_CHUNK), :],
                          gt_buf.at[dst, :], sem_g[b])
    hp = pltpu.async_copy(pre_hbm.at[img, pl.ds(r0, ROWS_PER_CHUNK), :],
                          pre_buf.at[dst, :], sem_p[b])
    return hg, hp

  handles = [None, None]
  handles[0] = issue(0)

  def zero_body(k, _):
    hist[pl.ds(k * LANES, LANES)] = zeros
    return 0
  lax.fori_loop(0, (LANES * BINS_PAD) // LANES, zero_body, 0)

  for c in range(N_CHUNKS):
    b = c % 2
    if c + 1 < N_CHUNKS:
      handles[(c + 1) % 2] = issue(c + 1)
    hg, hp = handles[b]
    hg.wait()
    hp.wait()
    boff = b * ROWS_PER_CHUNK

    @plsc.parallel_loop(0, VECS_PER_CHUNK, unroll=8)
    def vec_body(i, boff=boff):
      rr = boff + (i // VECS_PER_ROW)
      cc = (i % VECS_PER_ROW) * LANES
      g = gt_buf[rr, pl.ds(cc, LANES)]
      p = pre_buf[rr, pl.ds(cc, LANES)]
      idx = (g * NUM_CLASS + p) * LANES + lane
      plsc.addupdate_scatter(hist, [idx], ones)

  def scan_body(bb, _):
    v = hist[pl.ds(bb * LANES, LANES)]
    hist[pl.ds(bb * LANES, LANES)] = plsc.cumsum(v)
    return 0
  lax.fori_loop(0, BINS_PAD, scan_body, 0)

  def col_body(cc, _):
    idx = (cc * LANES + lane) * LANES + (LANES - 1)
    hist1d[pl.ds(cc * LANES, LANES)] = plsc.load_gather(hist, [idx])
    return 0
  lax.fori_loop(0, BINS_PAD // LANES, col_body, 0)

  pltpu.sync_copy(hist1d, out_hbm.at[wid])


@jax.jit
def _confusion(gt_img, pre_img):
  mesh = plsc.VectorSubcoreMesh(core_axis_name="c", subcore_axis_name="s")
  partials = pl.kernel(
      _sc_body,
      out_type=jax.ShapeDtypeStruct((NW, BINS_PAD), jnp.float32),
      mesh=mesh,
      compiler_params=pltpu.CompilerParams(needs_layout_passes=False),
      scratch_types=[
          pltpu.VMEM((2 * ROWS_PER_CHUNK, IMG_W), jnp.int32),
          pltpu.VMEM((2 * ROWS_PER_CHUNK, IMG_W), jnp.int32),
          pltpu.VMEM((LANES * BINS_PAD,), jnp.float32),
          pltpu.VMEM((BINS_PAD,), jnp.float32),
          pltpu.SemaphoreType.DMA,
          pltpu.SemaphoreType.DMA,
          pltpu.SemaphoreType.DMA,
          pltpu.SemaphoreType.DMA,
      ],
  )(gt_img, pre_img)
  return partials.sum(axis=0)[:NBINS].reshape(NUM_CLASS, NUM_CLASS)


def kernel(gt_image, pre_image):
  return _confusion(gt_image, pre_image)


# speedup vs baseline: 98.4259x; 1.0277x over previous
import functools

import jax
import jax.numpy as jnp
from jax import lax
from jax.experimental import pallas as pl
from jax.experimental.pallas import tpu as pltpu
from jax.experimental.pallas import tpu_sc as plsc

NUM_CLASS = 19
NBINS = NUM_CLASS * NUM_CLASS
BINS_PAD = 368
LANES = 16

N_TOTAL = 16 * 512 * 512
NC = 2
NS = 16
NW = NC * NS
IMG_H = 512
IMG_W = 512
ROWS_PER_W = 256
ROWS_PER_CHUNK = 32
CHUNK = ROWS_PER_CHUNK * IMG_W
N_CHUNKS = ROWS_PER_W // ROWS_PER_CHUNK
VECS_PER_CHUNK = CHUNK // LANES
VECS_PER_ROW = IMG_W // LANES


def _sc_body(gt_hbm, pre_hbm, out_hbm, gt_buf, pre_buf, hist, hist1d,
             sem_g0, sem_g1, sem_p0, sem_p1):
  wid = lax.axis_index("s") * NC + lax.axis_index("c")
  img = wid // 2
  row_base = (wid % 2) * ROWS_PER_W

  lane = jnp.arange(LANES, dtype=jnp.int32)
  ones = jnp.ones((LANES,), jnp.float32)
  zeros = jnp.zeros((LANES,), jnp.float32)

  sem_g = (sem_g0, sem_g1)
  sem_p = (sem_p0, sem_p1)

  def issue(c, b):
    r0 = row_base + c * ROWS_PER_CHUNK
    dst = pl.ds(b * ROWS_PER_CHUNK, ROWS_PER_CHUNK)
    hg = pltpu.async_copy(gt_hbm.at[img, pl.ds(r0, ROWS_PER_CHUNK), :],
                          gt_buf.at[dst, :], sem_g[b])
    hp = pltpu.async_copy(pre_hbm.at[img, pl.ds(r0, ROWS_PER_CHUNK), :],
                          pre_buf.at[dst, :], sem_p[b])
    return hg, hp

  def accumulate(b):
    boff = b * ROWS_PER_CHUNK

    @plsc.parallel_loop(0, VECS_PER_CHUNK, unroll=8)
    def vec_body(i):
      rr = boff + (i // VECS_PER_ROW)
      cc = (i % VECS_PER_ROW) * LANES
      g = gt_buf[rr, pl.ds(cc, LANES)]
      p = pre_buf[rr, pl.ds(cc, LANES)]
      idx = (g * NUM_CLASS + p) * LANES + lane
      plsc.addupdate_scatter(hist, [idx], ones)

  issue(0, 0)

  def zero_body(k, _):
    hist[pl.ds(k * LANES, LANES)] = zeros
    return 0
  lax.fori_loop(0, (LANES * BINS_PAD) // LANES, zero_body, 0)

  def pair_body(h, _):
    hg1, hp1 = issue(2 * h + 1, 1)
    pltpu.make_async_copy(
        gt_hbm.at[img, pl.ds(row_base, ROWS_PER_CHUNK), :],
        gt_buf.at[pl.ds(0, ROWS_PER_CHUNK), :], sem_g[0]).wait()
    pltpu.make_async_copy(
        pre_hbm.at[img, pl.ds(row_base, ROWS_PER_CHUNK), :],
        pre_buf.at[pl.ds(0, ROWS_PER_CHUNK), :], sem_p[0]).wait()
    accumulate(0)

    @pl.when(h < N_CHUNKS // 2 - 1)
    def _():
      issue(2 * h + 2, 0)

    hg1.wait()
    hp1.wait()
    accumulate(1)
    return 0
  lax.fori_loop(0, N_CHUNKS // 2, pair_body, 0)

  def scan_body(bb, _):
    v = hist[pl.ds(bb * LANES, LANES)]
    hist[pl.ds(bb * LANES, LANES)] = plsc.cumsum(v)
    return 0
  lax.fori_loop(0, BINS_PAD, scan_body, 0)

  def col_body(cc, _):
    idx = (cc * LANES + lane) * LANES + (LANES - 1)
    hist1d[pl.ds(cc * LANES, LANES)] = plsc.load_gather(hist, [idx])
    return 0
  lax.fori_loop(0, BINS_PAD // LANES, col_body, 0)

  pltpu.sync_copy(hist1d, out_hbm.at[wid])


@jax.jit
def _confusion(gt_img, pre_img):
  mesh = plsc.VectorSubcoreMesh(core_axis_name="c", subcore_axis_name="s")
  partials = pl.kernel(
      _sc_body,
      out_type=jax.ShapeDtypeStruct((NW, BINS_PAD), jnp.float32),
      mesh=mesh,
      compiler_params=pltpu.CompilerParams(needs_layout_passes=False),
      scratch_types=[
          pltpu.VMEM((2 * ROWS_PER_CHUNK, IMG_W), jnp.int32),
          pltpu.VMEM((2 * ROWS_PER_CHUNK, IMG_W), jnp.int32),
          pltpu.VMEM((LANES * BINS_PAD,), jnp.float32),
          pltpu.VMEM((BINS_PAD,), jnp.float32),
          pltpu.SemaphoreType.DMA,
          pltpu.SemaphoreType.DMA,
          pltpu.SemaphoreType.DMA,
          pltpu.SemaphoreType.DMA,
      ],
  )(gt_img, pre_img)
  return partials.sum(axis=0)[:NBINS].reshape(NUM_CLASS, NUM_CLASS)


def kernel(gt_image, pre_image):
  return _confusion(gt_image, pre_image)


# speedup vs baseline: 98.9022x; 1.0048x over previous
import functools

import jax
import jax.numpy as jnp
from jax import lax
from jax.experimental import pallas as pl
from jax.experimental.pallas import tpu as pltpu
from jax.experimental.pallas import tpu_sc as plsc

NUM_CLASS = 19
NBINS = NUM_CLASS * NUM_CLASS
BINS_PAD = 368
LANES = 16

N_TOTAL = 16 * 512 * 512
NC = 2
NS = 16
NW = NC * NS
IMG_H = 512
IMG_W = 512
ROWS_PER_W = 256
ROWS_PER_CHUNK = 32
CHUNK = ROWS_PER_CHUNK * IMG_W
N_CHUNKS = ROWS_PER_W // ROWS_PER_CHUNK
VECS_PER_CHUNK = CHUNK // LANES
VECS_PER_ROW = IMG_W // LANES


def _sc_body(gt_hbm, pre_hbm, out_hbm, gt_buf, pre_buf, hist, hist1d,
             sem_g, sem_p):
  wid = lax.axis_index("s") * NC + lax.axis_index("c")
  img = wid // 2
  row_base = (wid % 2) * ROWS_PER_W

  lane = jnp.arange(LANES, dtype=jnp.int32)
  ones = jnp.ones((LANES,), jnp.float32)
  zeros = jnp.zeros((LANES,), jnp.float32)

  def issue(c, b):
    r0 = row_base + c * ROWS_PER_CHUNK
    dst = pl.ds(b * ROWS_PER_CHUNK, ROWS_PER_CHUNK)
    pltpu.async_copy(gt_hbm.at[img, pl.ds(r0, ROWS_PER_CHUNK), :],
                     gt_buf.at[dst, :], sem_g.at[b])
    pltpu.async_copy(pre_hbm.at[img, pl.ds(r0, ROWS_PER_CHUNK), :],
                     pre_buf.at[dst, :], sem_p.at[b])

  issue(0, 0)

  def zero_body(k, _):
    hist[pl.ds(k * LANES, LANES)] = zeros
    return 0
  lax.fori_loop(0, (LANES * BINS_PAD) // LANES, zero_body, 0)

  def chunk_body(c, _):
    b = c % 2
    boff = b * ROWS_PER_CHUNK

    @pl.when(c + 1 < N_CHUNKS)
    def _():
      issue(c + 1, 1 - b)

    dst = pl.ds(boff, ROWS_PER_CHUNK)
    pltpu.make_async_copy(gt_hbm.at[img, pl.ds(row_base, ROWS_PER_CHUNK), :],
                          gt_buf.at[dst, :], sem_g.at[b]).wait()
    pltpu.make_async_copy(pre_hbm.at[img, pl.ds(row_base, ROWS_PER_CHUNK), :],
                          pre_buf.at[dst, :], sem_p.at[b]).wait()

    @plsc.parallel_loop(0, VECS_PER_CHUNK, unroll=8)
    def vec_body(i):
      rr = boff + (i // VECS_PER_ROW)
      cc = (i % VECS_PER_ROW) * LANES
      g = gt_buf[rr, pl.ds(cc, LANES)]
      p = pre_buf[rr, pl.ds(cc, LANES)]
      idx = (g * NUM_CLASS + p) * LANES + lane
      plsc.addupdate_scatter(hist, [idx], ones)
    return 0
  lax.fori_loop(0, N_CHUNKS, chunk_body, 0)

  def scan_body(bb, _):
    v = hist[pl.ds(bb * LANES, LANES)]
    hist[pl.ds(bb * LANES, LANES)] = plsc.cumsum(v)
    return 0
  lax.fori_loop(0, BINS_PAD, scan_body, 0)

  def col_body(cc, _):
    idx = (cc * LANES + lane) * LANES + (LANES - 1)
    hist1d[pl.ds(cc * LANES, LANES)] = plsc.load_gather(hist, [idx])
    return 0
  lax.fori_loop(0, BINS_PAD // LANES, col_body, 0)

  pltpu.sync_copy(hist1d, out_hbm.at[wid])


@jax.jit
def _confusion(gt_img, pre_img):
  mesh = plsc.VectorSubcoreMesh(core_axis_name="c", subcore_axis_name="s")
  partials = pl.kernel(
      _sc_body,
      out_type=jax.ShapeDtypeStruct((NW, BINS_PAD), jnp.float32),
      mesh=mesh,
      compiler_params=pltpu.CompilerParams(needs_layout_passes=False),
      scratch_types=[
          pltpu.VMEM((2 * ROWS_PER_CHUNK, IMG_W), jnp.int32),
          pltpu.VMEM((2 * ROWS_PER_CHUNK, IMG_W), jnp.int32),
          pltpu.VMEM((LANES * BINS_PAD,), jnp.float32),
          pltpu.VMEM((BINS_PAD,), jnp.float32),
          pltpu.SemaphoreType.DMA((2,)),
          pltpu.SemaphoreType.DMA((2,)),
      ],
  )(gt_img, pre_img)
  return partials.sum(axis=0)[:NBINS].reshape(NUM_CLASS, NUM_CLASS)


def kernel(gt_image, pre_image):
  return _confusion(gt_image, pre_image)
